# Initial kernel scaffold; baseline (speedup 1.0000x reference)
import jax
import jax.numpy as jnp
from jax.experimental import pallas as pl


def kernel(node_features, node_positions, edge_index, params):
    raise NotImplementedError("write your pallas kernel here")


# speedup vs baseline: 12.2471x; 12.2471x over previous
import functools

import jax
import jax.numpy as jnp
from jax import lax
from jax.experimental import pallas as pl
from jax.experimental.pallas import tpu as pltpu
from jax.experimental.pallas import tpu_sc as plsc

N_NODES = 50000
N_EDGES = 800000
D_S = 64
D_SM = 192
EDGE_FEAT = 20
CUTOFF = 5.0

NW = 32
NC = 2
NS = 16


def _sc_gather(table, idx, chunk, cj):
    total = idx.shape[0]
    d = table.shape[1]
    per_w = total // NW
    n_chunks = per_w // chunk
    j_rows = chunk // cj
    assert per_w * NW == total and n_chunks * chunk == per_w
    assert chunk % 8 == 0 and j_rows * cj == chunk and cj <= 128

    idx3 = idx.reshape(total // chunk, j_rows, cj)
    mesh = plsc.VectorSubcoreMesh(core_axis_name="c", subcore_axis_name="s")

    @functools.partial(
        pl.kernel,
        mesh=mesh,
        out_type=jax.ShapeDtypeStruct((total, d), jnp.float32),
        scratch_types=[
            pltpu.VMEM((j_rows, cj), jnp.int32),
            pltpu.VMEM((chunk, d), jnp.float32),
            pltpu.SemaphoreType.DMA,
        ],
        compiler_params=pltpu.CompilerParams(use_tc_tiling_on_sc=False),
    )
    def gk(table_hbm, idx_hbm, out_hbm, idx_v, rows_v, sem):
        wid = lax.axis_index("s") * NC + lax.axis_index("c")
        chunk0 = wid * n_chunks

        def body(k, _):
            gchunk = chunk0 + k
            pltpu.sync_copy(idx_hbm.at[gchunk], idx_v)
            descs = []
            for j in range(j_rows):
                descs.append(
                    pltpu.make_async_copy(
                        table_hbm.at[idx_v.at[j]],
                        rows_v.at[pl.ds(j * cj, cj)],
                        sem,
                    )
                )
            for dsc in descs:
                dsc.start()
            for dsc in descs:
                dsc.wait()
            base = gchunk * chunk
            pltpu.sync_copy(rows_v, out_hbm.at[pl.ds(base, chunk)])
            return 0

        lax.fori_loop(0, n_chunks, body, 0)

    return gk(table, idx3)


def _sc_scatter_add(x, dst, n_out, dc):
    e, d = x.shape
    chunk = 500
    assert e % (NW * chunk) == 0 and d % dc == 0 and n_out % NS == 0
    per_w = e // NW
    n_chunks = per_w // chunk
    n_pass = d // dc
    rows_per_sub = n_out // NS

    dst3 = dst.reshape(e // chunk, 4, 125)
    mesh = plsc.VectorSubcoreMesh(core_axis_name="c", subcore_axis_name="s")

    @functools.partial(
        pl.kernel,
        mesh=mesh,
        out_type=jax.ShapeDtypeStruct((n_out, NC, d), jnp.float32),
        scratch_types=[
            pltpu.VMEM_SHARED((n_out, dc), jnp.float32),
            pltpu.VMEM((4, 125), jnp.int32),
            pltpu.VMEM((chunk, dc), jnp.float32),
        ],
        compiler_params=pltpu.CompilerParams(use_tc_tiling_on_sc=False),
    )
    def sk(x_hbm, dst_hbm, zeros_hbm, out_hbm, acc, idx_v, x_v):
        cid = lax.axis_index("c")
        sid = lax.axis_index("s")
        row0 = sid * rows_per_sub
        chunk0 = cid * (NS * n_chunks) + sid * n_chunks

        def one_pass(p, _):
            d0 = p * dc
            pltpu.sync_copy(zeros_hbm, acc.at[pl.ds(row0, rows_per_sub)])
            plsc.subcore_barrier()

            def chunk_body(k, _):
                gchunk = chunk0 + k
                pltpu.sync_copy(dst_hbm.at[gchunk], idx_v)
                pltpu.sync_copy(
                    x_hbm.at[pl.ds(gchunk * chunk, chunk), pl.ds(d0, dc)], x_v
                )
                for j in range(4):
                    pltpu.sync_copy(
                        x_v.at[pl.ds(j * 125, 125)],
                        acc.at[idx_v.at[j]],
                        add=True,
                    )
                return 0

            lax.fori_loop(0, n_chunks, chunk_body, 0)
            plsc.subcore_barrier()
            pltpu.sync_copy(
                acc.at[pl.ds(row0, rows_per_sub)],
                out_hbm.at[pl.ds(row0, rows_per_sub), cid, pl.ds(d0, dc)],
            )
            plsc.subcore_barrier()
            return 0

        lax.fori_loop(0, n_pass, one_pass, 0)

    zeros = jnp.zeros((rows_per_sub, dc), jnp.float32)
    return sk(x, dst3, zeros)


def _row_specs(shapes, bn):
    specs = []
    for s, tiled in shapes:
        if tiled:
            blk = (bn,) + tuple(s[1:])
            nd = len(s)
            specs.append(
                pl.BlockSpec(blk, lambda i, _nd=nd: (i,) + (0,) * (_nd - 1))
            )
        else:
            specs.append(pl.BlockSpec(s, lambda i, _nd=len(s): (0,) * _nd))
    return specs


def _silu(x):
    return x * jax.nn.sigmoid(x)


def _tc_call(body, n_rows, bn, ins, in_tiled, out_shapes, out_tiled):
    grid = (n_rows // bn,)
    in_specs = _row_specs([(tuple(a.shape), t) for a, t in zip(ins, in_tiled)], bn)
    out_specs = _row_specs([(tuple(s.shape), t) for s, t in zip(out_shapes, out_tiled)], bn)
    return pl.pallas_call(
        body,
        grid=grid,
        in_specs=in_specs,
        out_specs=out_specs if len(out_shapes) > 1 else out_specs[0],
        out_shape=out_shapes if len(out_shapes) > 1 else out_shapes[0],
        compiler_params=pltpu.CompilerParams(
            dimension_semantics=("arbitrary",)
        ),
    )(*ins)


def _geometry(pos_src, pos_dst):
    e = pos_src.shape[0]
    bn = 2000

    def body(ps_ref, pd_ref, geo_ref):
        ps = ps_ref[...]
        pd = pd_ref[...]
        diff = pd - ps
        mask = lax.broadcasted_iota(jnp.int32, (1, 16), 1) < 3
        dm = jnp.where(mask, diff, 0.0)
        d2 = jnp.sum(dm * dm, axis=1, keepdims=True)
        dd = jnp.sqrt(d2 + 1e-12)
        unit = dm / (dd + 1e-10)
        fc = jnp.where(
            dd < CUTOFF, 0.5 * (jnp.cos(jnp.pi * dd / CUTOFF) + 1.0), 0.0
        )
        geo_ref[...] = jnp.concatenate(
            [unit[:, 0:3], dd, fc, jnp.zeros_like(dd), dd * 0.0, dd * 0.0],
            axis=1,
        )

    out = jax.ShapeDtypeStruct((e, 8), jnp.float32)
    return _tc_call(body, e, bn, [pos_src, pos_dst], [True, True], [out], [True])


def _node_msg(s, w1, b1, w2, b2):
    n = s.shape[0]
    bn = 2000

    def body(s_ref, w1_ref, b1_ref, w2_ref, b2_ref, o_ref):
        h = jnp.dot(s_ref[...], w1_ref[...], preferred_element_type=jnp.float32)
        h = _silu(h + b1_ref[...])
        o = jnp.dot(h, w2_ref[...], preferred_element_type=jnp.float32)
        o_ref[...] = o + b2_ref[...]

    out = jax.ShapeDtypeStruct((n, D_SM), jnp.float32)
    return _tc_call(
        body, n, bn,
        [s, w1, b1, w2, b2], [True, False, False, False, False],
        [out], [True],
    )


def _edge_kernel(geo8, sm_src, v_src, wf, bf):
    e = geo8.shape[0]
    bn = 2000

    def body(g_ref, sm_ref, v_ref, wf_ref, bf_ref, o_ref):
        ns = lax.broadcasted_iota(jnp.int32, (1, 24), 1).astype(jnp.float32) + 1.0
        geo = g_ref[...]
        dd = geo[:, 3:4]
        fc = geo[:, 4:5]
        es = jnp.sin(dd * ns * (jnp.pi / CUTOFF)) / dd
        fw = jnp.dot(es, wf_ref[...], preferred_element_type=jnp.float32)
        fw = (fw + bf_ref[...]) * fc
        fo = fw * sm_ref[...]
        g_vec = fo[:, 0:64]
        g_edge = fo[:, 64:128]
        g_sca = fo[:, 128:192]
        v = v_ref[...]
        parts = [g_sca]
        for k in range(3):
            uk = geo[:, k:k + 1]
            parts.append(v[:, 64 * k:64 * k + 64] * g_vec + g_edge * uk)
        o_ref[...] = jnp.concatenate(parts, axis=1)

    out = jax.ShapeDtypeStruct((e, 256), jnp.float32)
    return _tc_call(
        body, e, bn,
        [geo8, sm_src, v_src, wf, bf],
        [True, True, True, False, False],
        [out], [True],
    )


def _node_update(s, v, partials, wu, wv, w1, b1, w2, b2):
    n = s.shape[0]
    bn = 2000

    def body(s_ref, v_ref, p_ref, wu_ref, wv_ref, w1_ref, b1_ref, w2_ref,
             b2_ref, so_ref, vo_ref):
        agg = p_ref[:, 0, :] + p_ref[:, 1, :]
        s1 = s_ref[...] + agg[:, 0:64]
        v1 = v_ref[...] + agg[:, 64:256]
        wu_m = wu_ref[...]
        wv_m = wv_ref[...]
        uv = []
        vv = []
        for k in range(3):
            vk = v1[:, 64 * k:64 * k + 64]
            uv.append(jnp.dot(vk, wu_m, preferred_element_type=jnp.float32))
            vv.append(jnp.dot(vk, wv_m, preferred_element_type=jnp.float32))
        vn2 = vv[0] * vv[0] + vv[1] * vv[1] + vv[2] * vv[2]
        vn = jnp.sqrt(vn2 + 1e-8)
        h = jnp.concatenate([s1, vn], axis=1)
        h = jnp.dot(h, w1_ref[...], preferred_element_type=jnp.float32)
        h = _silu(h + b1_ref[...])
        m = jnp.dot(h, w2_ref[...], preferred_element_type=jnp.float32)
        m = m + b2_ref[...]
        a_vv = m[:, 0:64]
        a_sv = m[:, 64:128]
        a_ss = m[:, 128:192]
        v2 = [v1[:, 64 * k:64 * k + 64] + a_vv * uv[k] for k in range(3)]
        inner = uv[0] * vv[0] + uv[1] * vv[1] + uv[2] * vv[2]
        so_ref[...] = s1 + a_sv * inner + a_ss
        vo_ref[...] = jnp.concatenate(v2, axis=1)

    outs = [
        jax.ShapeDtypeStruct((n, D_S), jnp.float32),
        jax.ShapeDtypeStruct((n, D_SM), jnp.float32),
    ]
    return _tc_call(
        body, n, bn,
        [s, v, partials, wu, wv, w1, b1, w2, b2],
        [True, True, True, False, False, False, False, False, False],
        outs, [True, True],
    )


def _final_kernel(s, lap_partials, w1, b1, w2, b2):
    n = s.shape[0]
    bn = 2000

    def body(s_ref, p_ref, w1_ref, b1_ref, w2_ref, b2_ref, r_ref, lap_ref):
        sv = s_ref[...]
        h = jnp.dot(sv, w1_ref[...], preferred_element_type=jnp.float32)
        h = _silu(h + b1_ref[...])
        r = jnp.dot(h, w2_ref[...], preferred_element_type=jnp.float32)
        r_ref[...] = r + b2_ref[...]
        agg = p_ref[:, 0, :] + p_ref[:, 1, :]
        segsum = agg[:, 0:64]
        deg = agg[:, 64:65]
        lap_ref[...] = deg * sv - segsum

    outs = [
        jax.ShapeDtypeStruct((n, 1), jnp.float32),
        jax.ShapeDtypeStruct((n, D_S), jnp.float32),
    ]
    return _tc_call(
        body, n, bn,
        [s, lap_partials, w1, b1, w2, b2],
        [True, True, False, False, False, False],
        outs, [True, True],
    )


def kernel(node_features, node_positions, edge_index, params):
    n = node_positions.shape[0]
    e = edge_index.shape[1]
    src = edge_index[0].astype(jnp.int32)
    dst = edge_index[1].astype(jnp.int32)

    n_pad = 50176
    nf = jnp.pad(node_features.astype(jnp.int32), (0, n_pad - n))
    s = _sc_gather(params["embedding"], nf, chunk=224, cj=112)[:n]

    pos16 = jnp.pad(node_positions, ((0, 0), (0, 13)))
    p_src = _sc_gather(pos16, src, chunk=1000, cj=125)
    p_dst = _sc_gather(pos16, dst, chunk=1000, cj=125)
    geo8 = _geometry(p_src, p_dst)

    v = jnp.zeros((n, D_SM), jnp.float32)

    for bp in params["blocks"]:
        wf = jnp.pad(bp["filter"]["w"], ((0, 4), (0, 0)))
        bf = bp["filter"]["b"].reshape(1, D_SM)
        sm = _node_msg(
            s,
            bp["msg1"]["w"], bp["msg1"]["b"].reshape(1, D_S),
            bp["msg2"]["w"], bp["msg2"]["b"].reshape(1, D_SM),
        )
        sm_src = _sc_gather(sm, src, chunk=200, cj=100)
        v_src = _sc_gather(v, src, chunk=200, cj=100)
        scat_in = _edge_kernel(geo8, sm_src, v_src, wf, bf)
        partials = _sc_scatter_add(scat_in, dst, n, dc=32)
        s, v = _node_update(
            s, v, partials,
            bp["U"]["w"], bp["V"]["w"],
            bp["upd1"]["w"], bp["upd1"]["b"].reshape(1, D_S),
            bp["upd2"]["w"], bp["upd2"]["b"].reshape(1, D_SM),
        )

    s_ext = jnp.concatenate([s, jnp.ones((n, 16), jnp.float32)], axis=1)
    sd = _sc_gather(s_ext, dst, chunk=1000, cj=125)
    lap_partials = _sc_scatter_add(sd, src, n, dc=16)

    r0, r1 = params["readout"]
    r, lap = _final_kernel(
        s, lap_partials,
        r0["w"], r0["b"].reshape(1, D_S),
        r1["w"], r1["b"].reshape(1, 1),
    )
    return (r, s, lap)


# speedup vs baseline: 13.6167x; 1.1118x over previous
import functools

import jax
import jax.numpy as jnp
from jax import lax
from jax.experimental import pallas as pl
from jax.experimental.pallas import tpu as pltpu
from jax.experimental.pallas import tpu_sc as plsc

N_NODES = 50000
N_EDGES = 800000
D_S = 64
D_SM = 192
EDGE_FEAT = 20
CUTOFF = 5.0

NW = 32
NC = 2
NS = 16


def _sc_gather(table, idx, cj, rpc):
    total = idx.shape[0]
    d = table.shape[1]
    per_w = total // NW
    r_rows = per_w // cj
    chunk = rpc * cj
    n_chunks = r_rows // rpc
    n2 = n_chunks // 2
    assert per_w * NW == total and r_rows * cj == per_w
    assert n2 * 2 == n_chunks and cj <= 128

    idx2 = idx.reshape(total // cj, cj)
    mesh = plsc.VectorSubcoreMesh(core_axis_name="c", subcore_axis_name="s")

    @functools.partial(
        pl.kernel,
        mesh=mesh,
        out_type=jax.ShapeDtypeStruct((total, d), jnp.float32),
        scratch_types=[
            pltpu.VMEM((r_rows, cj), jnp.int32),
            pltpu.VMEM((chunk, d), jnp.float32),
            pltpu.VMEM((chunk, d), jnp.float32),
            pltpu.SemaphoreType.DMA,
            pltpu.SemaphoreType.DMA,
            pltpu.SemaphoreType.DMA,
            pltpu.SemaphoreType.DMA,
        ],
        compiler_params=pltpu.CompilerParams(use_tc_tiling_on_sc=False),
    )
    def gk(table_hbm, idx_hbm, out_hbm, idx_all, buf0, buf1, gs0, gs1,
           ws0, ws1):
        wid = lax.axis_index("s") * NC + lax.axis_index("c")
        base_w = wid * per_w
        pltpu.sync_copy(idx_hbm.at[pl.ds(wid * r_rows, r_rows)], idx_all)

        def fire_gather(k, buf, sem):
            for j in range(rpc):
                pltpu.async_copy(
                    table_hbm.at[idx_all.at[k * rpc + j]],
                    buf.at[pl.ds(j * cj, cj)],
                    sem,
                )

        def wait_gather(buf, sem):
            for j in range(rpc):
                pltpu.make_async_copy(
                    table_hbm.at[idx_all.at[j]], buf.at[pl.ds(j * cj, cj)],
                    sem,
                ).wait()

        def wr(k, buf, sem):
            return pltpu.make_async_copy(
                buf, out_hbm.at[pl.ds(base_w + k * chunk, chunk)], sem
            )

        def body(i, _):
            k0 = 2 * i
            k1 = 2 * i + 1

            @pl.when(i >= 1)
            def _():
                wr(0, buf0, ws0).wait()

            fire_gather(k0, buf0, gs0)

            @pl.when(i >= 1)
            def _():
                wr(0, buf1, ws1).wait()

            fire_gather(k1, buf1, gs1)
            wait_gather(buf0, gs0)
            wr(k0, buf0, ws0).start()
            wait_gather(buf1, gs1)
            wr(k1, buf1, ws1).start()
            return 0

        lax.fori_loop(0, n2, body, 0)
        wr(0, buf0, ws0).wait()
        wr(0, buf1, ws1).wait()

    return gk(table, idx2)


def _sc_scatter_add(x, dst, n_out, dc):
    e, d = x.shape
    chunk = 500
    rpc = 4
    cj = 125
    assert e % (NW * chunk) == 0 and d % dc == 0 and n_out % NS == 0
    per_w = e // NW
    r_rows = per_w // cj
    n_chunks = per_w // chunk
    n2 = n_chunks // 2
    assert n2 * 2 == n_chunks
    n_pass = d // dc
    rows_per_sub = n_out // NS

    dst2 = dst.reshape(e // cj, cj)
    mesh = plsc.VectorSubcoreMesh(core_axis_name="c", subcore_axis_name="s")

    @functools.partial(
        pl.kernel,
        mesh=mesh,
        out_type=jax.ShapeDtypeStruct((n_out, NC, d), jnp.float32),
        scratch_types=[
            pltpu.VMEM_SHARED((n_out, dc), jnp.float32),
            pltpu.VMEM((r_rows, cj), jnp.int32),
            pltpu.VMEM((chunk, dc), jnp.float32),
            pltpu.VMEM((chunk, dc), jnp.float32),
            pltpu.SemaphoreType.DMA,
            pltpu.SemaphoreType.DMA,
            pltpu.SemaphoreType.DMA,
            pltpu.SemaphoreType.DMA,
        ],
        compiler_params=pltpu.CompilerParams(use_tc_tiling_on_sc=False),
    )
    def sk(x_hbm, dst_hbm, zeros_hbm, out_hbm, acc, idx_all, xb0, xb1,
           xs0, xs1, ss0, ss1):
        cid = lax.axis_index("c")
        sid = lax.axis_index("s")
        row0 = sid * rows_per_sub
        wid = cid * NS + sid
        e0 = wid * per_w
        pltpu.sync_copy(dst_hbm.at[pl.ds(wid * r_rows, r_rows)], idx_all)

        def load_x(k, d0, buf, sem):
            pltpu.async_copy(
                x_hbm.at[pl.ds(e0 + k * chunk, chunk), pl.ds(d0, dc)],
                buf, sem,
            )

        def wait_x(d0, buf, sem):
            pltpu.make_async_copy(
                x_hbm.at[pl.ds(e0, chunk), pl.ds(d0, dc)], buf, sem
            ).wait()

        def fire_adds(k, buf, sem):
            for j in range(rpc):
                pltpu.async_copy(
                    buf.at[pl.ds(j * cj, cj)],
                    acc.at[idx_all.at[k * rpc + j]],
                    sem, add=True,
                )

        def wait_adds(buf, sem):
            for j in range(rpc):
                pltpu.make_async_copy(
                    buf.at[pl.ds(j * cj, cj)], acc.at[idx_all.at[j]], sem
                ).wait()

        def one_pass(p, _):
            d0 = p * dc
            pltpu.sync_copy(zeros_hbm, acc.at[pl.ds(row0, rows_per_sub)])
            plsc.subcore_barrier()

            def chunk_body(i, _):
                k0 = 2 * i
                k1 = 2 * i + 1

                @pl.when(i >= 1)
                def _():
                    wait_adds(xb0, ss0)

                load_x(k0, d0, xb0, xs0)

                @pl.when(i >= 1)
                def _():
                    wait_adds(xb1, ss1)

                load_x(k1, d0, xb1, xs1)
                wait_x(d0, xb0, xs0)
                fire_adds(k0, xb0, ss0)
                wait_x(d0, xb1, xs1)
                fire_adds(k1, xb1, ss1)
                return 0

            lax.fori_loop(0, n2, chunk_body, 0)
            wait_adds(xb0, ss0)
            wait_adds(xb1, ss1)
            plsc.subcore_barrier()
            pltpu.sync_copy(
                acc.at[pl.ds(row0, rows_per_sub)],
                out_hbm.at[pl.ds(row0, rows_per_sub), cid, pl.ds(d0, dc)],
            )
            plsc.subcore_barrier()
            return 0

        lax.fori_loop(0, n_pass, one_pass, 0)

    zeros = jnp.zeros((rows_per_sub, dc), jnp.float32)
    return sk(x, dst2, zeros)


def _row_specs(shapes, bn):
    specs = []
    for s, tiled in shapes:
        if tiled:
            blk = (bn,) + tuple(s[1:])
            nd = len(s)
            specs.append(
                pl.BlockSpec(blk, lambda i, _nd=nd: (i,) + (0,) * (_nd - 1))
            )
        else:
            specs.append(pl.BlockSpec(s, lambda i, _nd=len(s): (0,) * _nd))
    return specs


def _silu(x):
    return x * jax.nn.sigmoid(x)


def _tc_call(body, n_rows, bn, ins, in_tiled, out_shapes, out_tiled):
    grid = (n_rows // bn,)
    in_specs = _row_specs([(tuple(a.shape), t) for a, t in zip(ins, in_tiled)], bn)
    out_specs = _row_specs([(tuple(s.shape), t) for s, t in zip(out_shapes, out_tiled)], bn)
    return pl.pallas_call(
        body,
        grid=grid,
        in_specs=in_specs,
        out_specs=out_specs if len(out_shapes) > 1 else out_specs[0],
        out_shape=out_shapes if len(out_shapes) > 1 else out_shapes[0],
        compiler_params=pltpu.CompilerParams(
            dimension_semantics=("arbitrary",)
        ),
    )(*ins)


def _geometry(pos_src, pos_dst):
    e = pos_src.shape[0]
    bn = 2000

    def body(ps_ref, pd_ref, geo_ref):
        ps = ps_ref[...]
        pd = pd_ref[...]
        diff = pd - ps
        mask = lax.broadcasted_iota(jnp.int32, (1, 16), 1) < 3
        dm = jnp.where(mask, diff, 0.0)
        d2 = jnp.sum(dm * dm, axis=1, keepdims=True)
        dd = jnp.sqrt(d2 + 1e-12)
        unit = dm / (dd + 1e-10)
        fc = jnp.where(
            dd < CUTOFF, 0.5 * (jnp.cos(jnp.pi * dd / CUTOFF) + 1.0), 0.0
        )
        geo_ref[...] = jnp.concatenate(
            [unit[:, 0:3], dd, fc, jnp.zeros_like(dd), dd * 0.0, dd * 0.0],
            axis=1,
        )

    out = jax.ShapeDtypeStruct((e, 8), jnp.float32)
    return _tc_call(body, e, bn, [pos_src, pos_dst], [True, True], [out], [True])


def _node_msg(s, w1, b1, w2, b2):
    n = s.shape[0]
    bn = 2000

    def body(s_ref, w1_ref, b1_ref, w2_ref, b2_ref, o_ref):
        h = jnp.dot(s_ref[...], w1_ref[...], preferred_element_type=jnp.float32)
        h = _silu(h + b1_ref[...])
        o = jnp.dot(h, w2_ref[...], preferred_element_type=jnp.float32)
        o_ref[...] = o + b2_ref[...]

    out = jax.ShapeDtypeStruct((n, D_SM), jnp.float32)
    return _tc_call(
        body, n, bn,
        [s, w1, b1, w2, b2], [True, False, False, False, False],
        [out], [True],
    )


def _edge_kernel(geo8, sm_src, v_src, wf, bf):
    e = geo8.shape[0]
    bn = 2000

    def body(g_ref, sm_ref, v_ref, wf_ref, bf_ref, o_ref):
        ns = lax.broadcasted_iota(jnp.int32, (1, 24), 1).astype(jnp.float32) + 1.0
        geo = g_ref[...]
        dd = geo[:, 3:4]
        fc = geo[:, 4:5]
        es = jnp.sin(dd * ns * (jnp.pi / CUTOFF)) / dd
        fw = jnp.dot(es, wf_ref[...], preferred_element_type=jnp.float32)
        fw = (fw + bf_ref[...]) * fc
        fo = fw * sm_ref[...]
        g_vec = fo[:, 0:64]
        g_edge = fo[:, 64:128]
        g_sca = fo[:, 128:192]
        v = v_ref[...]
        parts = [g_sca]
        for k in range(3):
            uk = geo[:, k:k + 1]
            parts.append(v[:, 64 * k:64 * k + 64] * g_vec + g_edge * uk)
        o_ref[...] = jnp.concatenate(parts, axis=1)

    out = jax.ShapeDtypeStruct((e, 256), jnp.float32)
    return _tc_call(
        body, e, bn,
        [geo8, sm_src, v_src, wf, bf],
        [True, True, True, False, False],
        [out], [True],
    )


def _edge_kernel0(geo8, sm_src, wf, bf):
    e = geo8.shape[0]
    bn = 2000

    def body(g_ref, sm_ref, wf_ref, bf_ref, o_ref):
        ns = lax.broadcasted_iota(jnp.int32, (1, 24), 1).astype(jnp.float32) + 1.0
        geo = g_ref[...]
        dd = geo[:, 3:4]
        fc = geo[:, 4:5]
        es = jnp.sin(dd * ns * (jnp.pi / CUTOFF)) / dd
        fw = jnp.dot(es, wf_ref[...], preferred_element_type=jnp.float32)
        fw = (fw + bf_ref[...]) * fc
        fo = fw * sm_ref[...]
        g_edge = fo[:, 64:128]
        g_sca = fo[:, 128:192]
        parts = [g_sca]
        for k in range(3):
            parts.append(g_edge * geo[:, k:k + 1])
        o_ref[...] = jnp.concatenate(parts, axis=1)

    out = jax.ShapeDtypeStruct((e, 256), jnp.float32)
    return _tc_call(
        body, e, bn,
        [geo8, sm_src, wf, bf],
        [True, True, False, False],
        [out], [True],
    )


def _node_update(s, v, partials, wu, wv, w1, b1, w2, b2):
    n = s.shape[0]
    bn = 2000

    def body(s_ref, v_ref, p_ref, wu_ref, wv_ref, w1_ref, b1_ref, w2_ref,
             b2_ref, so_ref, vo_ref):
        agg = p_ref[:, 0, :] + p_ref[:, 1, :]
        s1 = s_ref[...] + agg[:, 0:64]
        v1 = v_ref[...] + agg[:, 64:256]
        wu_m = wu_ref[...]
        wv_m = wv_ref[...]
        uv = []
        vv = []
        for k in range(3):
            vk = v1[:, 64 * k:64 * k + 64]
            uv.append(jnp.dot(vk, wu_m, preferred_element_type=jnp.float32))
            vv.append(jnp.dot(vk, wv_m, preferred_element_type=jnp.float32))
        vn2 = vv[0] * vv[0] + vv[1] * vv[1] + vv[2] * vv[2]
        vn = jnp.sqrt(vn2 + 1e-8)
        h = jnp.concatenate([s1, vn], axis=1)
        h = jnp.dot(h, w1_ref[...], preferred_element_type=jnp.float32)
        h = _silu(h + b1_ref[...])
        m = jnp.dot(h, w2_ref[...], preferred_element_type=jnp.float32)
        m = m + b2_ref[...]
        a_vv = m[:, 0:64]
        a_sv = m[:, 64:128]
        a_ss = m[:, 128:192]
        v2 = [v1[:, 64 * k:64 * k + 64] + a_vv * uv[k] for k in range(3)]
        inner = uv[0] * vv[0] + uv[1] * vv[1] + uv[2] * vv[2]
        so_ref[...] = s1 + a_sv * inner + a_ss
        vo_ref[...] = jnp.concatenate(v2, axis=1)

    outs = [
        jax.ShapeDtypeStruct((n, D_S), jnp.float32),
        jax.ShapeDtypeStruct((n, D_SM), jnp.float32),
    ]
    return _tc_call(
        body, n, bn,
        [s, v, partials, wu, wv, w1, b1, w2, b2],
        [True, True, True, False, False, False, False, False, False],
        outs, [True, True],
    )


def _final_kernel(s, lap_partials, w1, b1, w2, b2):
    n = s.shape[0]
    bn = 2000

    def body(s_ref, p_ref, w1_ref, b1_ref, w2_ref, b2_ref, r_ref, lap_ref):
        sv = s_ref[...]
        h = jnp.dot(sv, w1_ref[...], preferred_element_type=jnp.float32)
        h = _silu(h + b1_ref[...])
        r = jnp.dot(h, w2_ref[...], preferred_element_type=jnp.float32)
        r_ref[...] = r + b2_ref[...]
        agg = p_ref[:, 0, :] + p_ref[:, 1, :]
        segsum = agg[:, 0:64]
        deg = agg[:, 64:65]
        lap_ref[...] = deg * sv - segsum

    outs = [
        jax.ShapeDtypeStruct((n, 1), jnp.float32),
        jax.ShapeDtypeStruct((n, D_S), jnp.float32),
    ]
    return _tc_call(
        body, n, bn,
        [s, lap_partials, w1, b1, w2, b2],
        [True, True, False, False, False, False],
        outs, [True, True],
    )


def kernel(node_features, node_positions, edge_index, params):
    n = node_positions.shape[0]
    e = edge_index.shape[1]
    src = edge_index[0].astype(jnp.int32)
    dst = edge_index[1].astype(jnp.int32)

    n_pad = 50176
    nf = jnp.pad(node_features.astype(jnp.int32), (0, n_pad - n))
    s = _sc_gather(params["embedding"], nf, cj=112, rpc=7)[:n]

    pos16 = jnp.pad(node_positions, ((0, 0), (0, 13)))
    p_src = _sc_gather(pos16, src, cj=125, rpc=4)
    p_dst = _sc_gather(pos16, dst, cj=125, rpc=4)
    geo8 = _geometry(p_src, p_dst)

    v = jnp.zeros((n, D_SM), jnp.float32)

    for bi, bp in enumerate(params["blocks"]):
        wf = jnp.pad(bp["filter"]["w"], ((0, 4), (0, 0)))
        bf = bp["filter"]["b"].reshape(1, D_SM)
        sm = _node_msg(
            s,
            bp["msg1"]["w"], bp["msg1"]["b"].reshape(1, D_S),
            bp["msg2"]["w"], bp["msg2"]["b"].reshape(1, D_SM),
        )
        sm_src = _sc_gather(sm, src, cj=125, rpc=2)
        if bi == 0:
            scat_in = _edge_kernel0(geo8, sm_src, wf, bf)
        else:
            v_src = _sc_gather(v, src, cj=125, rpc=2)
            scat_in = _edge_kernel(geo8, sm_src, v_src, wf, bf)
        partials = _sc_scatter_add(scat_in, dst, n, dc=16)
        s, v = _node_update(
            s, v, partials,
            bp["U"]["w"], bp["V"]["w"],
            bp["upd1"]["w"], bp["upd1"]["b"].reshape(1, D_S),
            bp["upd2"]["w"], bp["upd2"]["b"].reshape(1, D_SM),
        )

    s_ext = jnp.concatenate([s, jnp.ones((n, 16), jnp.float32)], axis=1)
    sd = _sc_gather(s_ext, dst, cj=125, rpc=4)
    lap_partials = _sc_scatter_add(sd, src, n, dc=16)

    r0, r1 = params["readout"]
    r, lap = _final_kernel(
        s, lap_partials,
        r0["w"], r0["b"].reshape(1, D_S),
        r1["w"], r1["b"].reshape(1, 1),
    )
    return (r, s, lap)


# speedup vs baseline: 13.7720x; 1.0114x over previous
import functools

import jax
import jax.numpy as jnp
from jax import lax
from jax.experimental import pallas as pl
from jax.experimental.pallas import tpu as pltpu
from jax.experimental.pallas import tpu_sc as plsc

N_NODES = 50000
N_EDGES = 800000
D_S = 64
D_SM = 192
EDGE_FEAT = 20
CUTOFF = 5.0

NW = 32
NC = 2
NS = 16


def _sc_gather(table, idx, cj, rpc):
    total = idx.shape[0]
    d = table.shape[1]
    per_w = total // NW
    r_rows = per_w // cj
    chunk = rpc * cj
    n_chunks = r_rows // rpc
    n2 = n_chunks // 2
    assert per_w * NW == total and r_rows * cj == per_w
    assert n2 * 2 == n_chunks and cj <= 128

    idx2 = idx.reshape(total // cj, cj)
    mesh = plsc.VectorSubcoreMesh(core_axis_name="c", subcore_axis_name="s")

    @functools.partial(
        pl.kernel,
        mesh=mesh,
        out_type=jax.ShapeDtypeStruct((total, d), jnp.float32),
        scratch_types=[
            pltpu.VMEM((r_rows, cj), jnp.int32),
            pltpu.VMEM((chunk, d), jnp.float32),
            pltpu.VMEM((chunk, d), jnp.float32),
            pltpu.SemaphoreType.DMA,
            pltpu.SemaphoreType.DMA,
            pltpu.SemaphoreType.DMA,
            pltpu.SemaphoreType.DMA,
        ],
        compiler_params=pltpu.CompilerParams(use_tc_tiling_on_sc=False),
    )
    def gk(table_hbm, idx_hbm, out_hbm, idx_all, buf0, buf1, gs0, gs1,
           ws0, ws1):
        wid = lax.axis_index("s") * NC + lax.axis_index("c")
        base_w = wid * per_w
        pltpu.sync_copy(idx_hbm.at[pl.ds(wid * r_rows, r_rows)], idx_all)

        def fire_gather(k, buf, sem):
            for j in range(rpc):
                pltpu.async_copy(
                    table_hbm.at[idx_all.at[k * rpc + j]],
                    buf.at[pl.ds(j * cj, cj)],
                    sem,
                )

        def wait_gather(buf, sem):
            for j in range(rpc):
                pltpu.make_async_copy(
                    table_hbm.at[idx_all.at[j]], buf.at[pl.ds(j * cj, cj)],
                    sem,
                ).wait()

        def wr(k, buf, sem):
            return pltpu.make_async_copy(
                buf, out_hbm.at[pl.ds(base_w + k * chunk, chunk)], sem
            )

        def body(i, _):
            k0 = 2 * i
            k1 = 2 * i + 1

            @pl.when(i >= 1)
            def _():
                wr(0, buf0, ws0).wait()

            fire_gather(k0, buf0, gs0)

            @pl.when(i >= 1)
            def _():
                wr(0, buf1, ws1).wait()

            fire_gather(k1, buf1, gs1)
            wait_gather(buf0, gs0)
            wr(k0, buf0, ws0).start()
            wait_gather(buf1, gs1)
            wr(k1, buf1, ws1).start()
            return 0

        lax.fori_loop(0, n2, body, 0)
        wr(0, buf0, ws0).wait()
        wr(0, buf1, ws1).wait()

    return gk(table, idx2)


def _sc_scatter_add(x, dst, n_out, dc, tiled=False):
    e, d = x.shape
    chunk = 1000 if tiled else 500
    rpc = 8 if tiled else 4
    cj = 125
    assert e % (NW * chunk) == 0 and d % dc == 0 and n_out % NS == 0
    per_w = e // NW
    r_rows = per_w // cj
    n_chunks = per_w // chunk
    odd = n_chunks % 2
    n2 = (n_chunks - odd) // 2
    n_pass = d // dc
    rows_per_sub = n_out // NS
    if tiled:
        assert rows_per_sub % 8 == 0 and chunk % 8 == 0

    dst2 = dst.reshape(e // cj, cj)
    mesh = plsc.VectorSubcoreMesh(core_axis_name="c", subcore_axis_name="s")

    @functools.partial(
        pl.kernel,
        mesh=mesh,
        out_type=jax.ShapeDtypeStruct((NC, n_out, d), jnp.float32),
        scratch_types=[
            pltpu.VMEM_SHARED((n_out, dc), jnp.float32),
            pltpu.VMEM((r_rows, cj), jnp.int32),
            pltpu.VMEM((chunk, dc), jnp.float32),
            pltpu.VMEM((chunk, dc), jnp.float32),
            pltpu.SemaphoreType.DMA,
            pltpu.SemaphoreType.DMA,
            pltpu.SemaphoreType.DMA,
            pltpu.SemaphoreType.DMA,
        ],
        compiler_params=pltpu.CompilerParams(use_tc_tiling_on_sc=tiled),
    )
    def sk(x_hbm, dst_hbm, zeros_hbm, out_hbm, acc, idx_all, xb0, xb1,
           xs0, xs1, ss0, ss1):
        cid = lax.axis_index("c")
        sid = lax.axis_index("s")
        row0 = sid * rows_per_sub
        wid = cid * NS + sid
        e0 = wid * per_w
        pltpu.sync_copy(dst_hbm.at[pl.ds(wid * r_rows, r_rows)], idx_all)

        def load_x(k, d0, buf, sem):
            pltpu.async_copy(
                x_hbm.at[pl.ds(e0 + k * chunk, chunk), pl.ds(d0, dc)],
                buf, sem,
            )

        def wait_x(d0, buf, sem):
            pltpu.make_async_copy(
                x_hbm.at[pl.ds(e0, chunk), pl.ds(d0, dc)], buf, sem
            ).wait()

        def fire_adds(k, buf, sem):
            for j in range(rpc):
                pltpu.async_copy(
                    buf.at[pl.ds(j * cj, cj)],
                    acc.at[idx_all.at[k * rpc + j]],
                    sem, add=True,
                )

        def wait_adds(buf, sem):
            for j in range(rpc):
                pltpu.make_async_copy(
                    buf.at[pl.ds(j * cj, cj)], acc.at[idx_all.at[j]], sem
                ).wait()

        def one_pass(p, _):
            d0 = p * dc
            pltpu.sync_copy(zeros_hbm, acc.at[pl.ds(row0, rows_per_sub)])
            plsc.subcore_barrier()

            if odd:
                load_x(0, d0, xb0, xs0)
                wait_x(d0, xb0, xs0)
                fire_adds(0, xb0, ss0)
                wait_adds(xb0, ss0)

            def chunk_body(i, _):
                k0 = odd + 2 * i
                k1 = k0 + 1

                @pl.when(i >= 1)
                def _():
                    wait_adds(xb0, ss0)

                load_x(k0, d0, xb0, xs0)

                @pl.when(i >= 1)
                def _():
                    wait_adds(xb1, ss1)

                load_x(k1, d0, xb1, xs1)
                wait_x(d0, xb0, xs0)
                fire_adds(k0, xb0, ss0)
                wait_x(d0, xb1, xs1)
                fire_adds(k1, xb1, ss1)
                return 0

            lax.fori_loop(0, n2, chunk_body, 0)
            wait_adds(xb0, ss0)
            wait_adds(xb1, ss1)
            plsc.subcore_barrier()
            pltpu.sync_copy(
                acc.at[pl.ds(row0, rows_per_sub)],
                out_hbm.at[cid, pl.ds(row0, rows_per_sub), pl.ds(d0, dc)],
            )
            plsc.subcore_barrier()
            return 0

        lax.fori_loop(0, n_pass, one_pass, 0)

    zeros = jnp.zeros((rows_per_sub, dc), jnp.float32)
    return sk(x, dst2, zeros)


def _row_specs(shapes, bn):
    specs = []
    for s, tiled in shapes:
        if tiled == "p":
            specs.append(
                pl.BlockSpec((s[0], bn, s[2]), lambda i: (0, i, 0))
            )
        elif tiled:
            blk = (bn,) + tuple(s[1:])
            nd = len(s)
            specs.append(
                pl.BlockSpec(blk, lambda i, _nd=nd: (i,) + (0,) * (_nd - 1))
            )
        else:
            specs.append(pl.BlockSpec(s, lambda i, _nd=len(s): (0,) * _nd))
    return specs


def _silu(x):
    return x * jax.nn.sigmoid(x)


def _tc_call(body, n_rows, bn, ins, in_tiled, out_shapes, out_tiled):
    grid = (n_rows // bn,)
    in_specs = _row_specs([(tuple(a.shape), t) for a, t in zip(ins, in_tiled)], bn)
    out_specs = _row_specs([(tuple(s.shape), t) for s, t in zip(out_shapes, out_tiled)], bn)
    return pl.pallas_call(
        body,
        grid=grid,
        in_specs=in_specs,
        out_specs=out_specs if len(out_shapes) > 1 else out_specs[0],
        out_shape=out_shapes if len(out_shapes) > 1 else out_shapes[0],
        compiler_params=pltpu.CompilerParams(
            dimension_semantics=("arbitrary",)
        ),
    )(*ins)


def _geometry(pos_src, pos_dst):
    e = pos_src.shape[0]
    bn = 2000

    def body(ps_ref, pd_ref, geo_ref):
        ps = ps_ref[...]
        pd = pd_ref[...]
        diff = pd - ps
        mask = lax.broadcasted_iota(jnp.int32, (1, 16), 1) < 3
        dm = jnp.where(mask, diff, 0.0)
        d2 = jnp.sum(dm * dm, axis=1, keepdims=True)
        dd = jnp.sqrt(d2 + 1e-12)
        unit = dm / (dd + 1e-10)
        fc = jnp.where(
            dd < CUTOFF, 0.5 * (jnp.cos(jnp.pi * dd / CUTOFF) + 1.0), 0.0
        )
        geo_ref[...] = jnp.concatenate(
            [unit[:, 0:3], dd, fc, jnp.zeros_like(dd), dd * 0.0, dd * 0.0],
            axis=1,
        )

    out = jax.ShapeDtypeStruct((e, 8), jnp.float32)
    return _tc_call(body, e, bn, [pos_src, pos_dst], [True, True], [out], [True])


def _node_msg(s, w1, b1, w2, b2):
    n = s.shape[0]
    bn = 2000

    def body(s_ref, w1_ref, b1_ref, w2_ref, b2_ref, o_ref):
        h = jnp.dot(s_ref[...], w1_ref[...], preferred_element_type=jnp.float32)
        h = _silu(h + b1_ref[...])
        o = jnp.dot(h, w2_ref[...], preferred_element_type=jnp.float32)
        o_ref[...] = o + b2_ref[...]

    out = jax.ShapeDtypeStruct((n, D_SM), jnp.float32)
    return _tc_call(
        body, n, bn,
        [s, w1, b1, w2, b2], [True, False, False, False, False],
        [out], [True],
    )


def _edge_kernel(geo8, sm_src, v_src, wf, bf):
    e = geo8.shape[0]
    bn = 2000

    def body(g_ref, sm_ref, v_ref, wf_ref, bf_ref, o_ref):
        ns = lax.broadcasted_iota(jnp.int32, (1, 24), 1).astype(jnp.float32) + 1.0
        geo = g_ref[...]
        dd = geo[:, 3:4]
        fc = geo[:, 4:5]
        es = jnp.sin(dd * ns * (jnp.pi / CUTOFF)) / dd
        fw = jnp.dot(es, wf_ref[...], preferred_element_type=jnp.float32)
        fw = (fw + bf_ref[...]) * fc
        fo = fw * sm_ref[...]
        g_vec = fo[:, 0:64]
        g_edge = fo[:, 64:128]
        g_sca = fo[:, 128:192]
        v = v_ref[...]
        parts = [g_sca]
        for k in range(3):
            uk = geo[:, k:k + 1]
            parts.append(v[:, 64 * k:64 * k + 64] * g_vec + g_edge * uk)
        o_ref[...] = jnp.concatenate(parts, axis=1)

    out = jax.ShapeDtypeStruct((e, 256), jnp.float32)
    return _tc_call(
        body, e, bn,
        [geo8, sm_src, v_src, wf, bf],
        [True, True, True, False, False],
        [out], [True],
    )


def _edge_kernel0(geo8, sm_src, wf, bf):
    e = geo8.shape[0]
    bn = 2000

    def body(g_ref, sm_ref, wf_ref, bf_ref, o_ref):
        ns = lax.broadcasted_iota(jnp.int32, (1, 24), 1).astype(jnp.float32) + 1.0
        geo = g_ref[...]
        dd = geo[:, 3:4]
        fc = geo[:, 4:5]
        es = jnp.sin(dd * ns * (jnp.pi / CUTOFF)) / dd
        fw = jnp.dot(es, wf_ref[...], preferred_element_type=jnp.float32)
        fw = (fw + bf_ref[...]) * fc
        fo = fw * sm_ref[...]
        g_edge = fo[:, 64:128]
        g_sca = fo[:, 128:192]
        parts = [g_sca]
        for k in range(3):
            parts.append(g_edge * geo[:, k:k + 1])
        o_ref[...] = jnp.concatenate(parts, axis=1)

    out = jax.ShapeDtypeStruct((e, 256), jnp.float32)
    return _tc_call(
        body, e, bn,
        [geo8, sm_src, wf, bf],
        [True, True, False, False],
        [out], [True],
    )


def _node_update(s, v, partials, wu, wv, w1, b1, w2, b2):
    n = s.shape[0]
    bn = 2000

    def body(s_ref, v_ref, p_ref, wu_ref, wv_ref, w1_ref, b1_ref, w2_ref,
             b2_ref, so_ref, vo_ref):
        agg = p_ref[0] + p_ref[1]
        s1 = s_ref[...] + agg[:, 0:64]
        v1 = v_ref[...] + agg[:, 64:256]
        wu_m = wu_ref[...]
        wv_m = wv_ref[...]
        uv = []
        vv = []
        for k in range(3):
            vk = v1[:, 64 * k:64 * k + 64]
            uv.append(jnp.dot(vk, wu_m, preferred_element_type=jnp.float32))
            vv.append(jnp.dot(vk, wv_m, preferred_element_type=jnp.float32))
        vn2 = vv[0] * vv[0] + vv[1] * vv[1] + vv[2] * vv[2]
        vn = jnp.sqrt(vn2 + 1e-8)
        h = jnp.concatenate([s1, vn], axis=1)
        h = jnp.dot(h, w1_ref[...], preferred_element_type=jnp.float32)
        h = _silu(h + b1_ref[...])
        m = jnp.dot(h, w2_ref[...], preferred_element_type=jnp.float32)
        m = m + b2_ref[...]
        a_vv = m[:, 0:64]
        a_sv = m[:, 64:128]
        a_ss = m[:, 128:192]
        v2 = [v1[:, 64 * k:64 * k + 64] + a_vv * uv[k] for k in range(3)]
        inner = uv[0] * vv[0] + uv[1] * vv[1] + uv[2] * vv[2]
        so_ref[...] = s1 + a_sv * inner + a_ss
        vo_ref[...] = jnp.concatenate(v2, axis=1)

    outs = [
        jax.ShapeDtypeStruct((n, D_S), jnp.float32),
        jax.ShapeDtypeStruct((n, D_SM), jnp.float32),
    ]
    return _tc_call(
        body, n, bn,
        [s, v, partials, wu, wv, w1, b1, w2, b2],
        [True, True, "p", False, False, False, False, False, False],
        outs, [True, True],
    )


def _final_kernel(s, lap_partials, w1, b1, w2, b2):
    n = s.shape[0]
    bn = 2000

    def body(s_ref, p_ref, w1_ref, b1_ref, w2_ref, b2_ref, r_ref, lap_ref):
        sv = s_ref[...]
        h = jnp.dot(sv, w1_ref[...], preferred_element_type=jnp.float32)
        h = _silu(h + b1_ref[...])
        r = jnp.dot(h, w2_ref[...], preferred_element_type=jnp.float32)
        r_ref[...] = r + b2_ref[...]
        agg = p_ref[0] + p_ref[1]
        segsum = agg[:, 0:64]
        deg = agg[:, 64:65]
        lap_ref[...] = deg * sv - segsum

    outs = [
        jax.ShapeDtypeStruct((n, 1), jnp.float32),
        jax.ShapeDtypeStruct((n, D_S), jnp.float32),
    ]
    return _tc_call(
        body, n, bn,
        [s, lap_partials, w1, b1, w2, b2],
        [True, "p", False, False, False, False],
        outs, [True, True],
    )


def kernel(node_features, node_positions, edge_index, params):
    n = node_positions.shape[0]
    e = edge_index.shape[1]
    src = edge_index[0].astype(jnp.int32)
    dst = edge_index[1].astype(jnp.int32)

    n_pad = 50176
    nf = jnp.pad(node_features.astype(jnp.int32), (0, n_pad - n))
    s = _sc_gather(params["embedding"], nf, cj=112, rpc=7)[:n]

    pos16 = jnp.pad(node_positions, ((0, 0), (0, 13)))
    p_src = _sc_gather(pos16, src, cj=125, rpc=4)
    p_dst = _sc_gather(pos16, dst, cj=125, rpc=4)
    geo8 = _geometry(p_src, p_dst)

    v = jnp.zeros((n, D_SM), jnp.float32)

    for bi, bp in enumerate(params["blocks"]):
        wf = jnp.pad(bp["filter"]["w"], ((0, 4), (0, 0)))
        bf = bp["filter"]["b"].reshape(1, D_SM)
        sm = _node_msg(
            s,
            bp["msg1"]["w"], bp["msg1"]["b"].reshape(1, D_S),
            bp["msg2"]["w"], bp["msg2"]["b"].reshape(1, D_SM),
        )
        sm_src = _sc_gather(sm, src, cj=125, rpc=2)
        if bi == 0:
            scat_in = _edge_kernel0(geo8, sm_src, wf, bf)
        else:
            v_src = _sc_gather(v, src, cj=125, rpc=2)
            scat_in = _edge_kernel(geo8, sm_src, v_src, wf, bf)
        partials = _sc_scatter_add(scat_in, dst, n, dc=16)
        s, v = _node_update(
            s, v, partials,
            bp["U"]["w"], bp["V"]["w"],
            bp["upd1"]["w"], bp["upd1"]["b"].reshape(1, D_S),
            bp["upd2"]["w"], bp["upd2"]["b"].reshape(1, D_SM),
        )

    s_ext = jnp.concatenate([s, jnp.ones((n, 16), jnp.float32)], axis=1)
    sd = _sc_gather(s_ext, dst, cj=125, rpc=4)
    lap_partials = _sc_scatter_add(sd, src, n, dc=16)

    r0, r1 = params["readout"]
    r, lap = _final_kernel(
        s, lap_partials,
        r0["w"], r0["b"].reshape(1, D_S),
        r1["w"], r1["b"].reshape(1, 1),
    )
    return (r, s, lap)


# speedup vs baseline: 15.0711x; 1.0943x over previous
import functools

import jax
import jax.numpy as jnp
from jax import lax
from jax.experimental import pallas as pl
from jax.experimental.pallas import tpu as pltpu
from jax.experimental.pallas import tpu_sc as plsc

N_NODES = 50000
N_EDGES = 800000
D_S = 64
D_SM = 192
EDGE_FEAT = 20
CUTOFF = 5.0

NW = 32
NC = 2
NS = 16


def _sc_gather(table, idx, cj, rpc):
    total = idx.shape[0]
    d = table.shape[1]
    per_w = total // NW
    r_rows = per_w // cj
    chunk = rpc * cj
    n_chunks = r_rows // rpc
    n2 = n_chunks // 2
    assert per_w * NW == total and r_rows * cj == per_w
    assert n2 * 2 == n_chunks and cj <= 128

    idx2 = idx.reshape(total // cj, cj)
    mesh = plsc.VectorSubcoreMesh(core_axis_name="c", subcore_axis_name="s")

    @functools.partial(
        pl.kernel,
        mesh=mesh,
        out_type=jax.ShapeDtypeStruct((total, d), jnp.float32),
        scratch_types=[
            pltpu.VMEM((r_rows, cj), jnp.int32),
            pltpu.VMEM((chunk, d), jnp.float32),
            pltpu.VMEM((chunk, d), jnp.float32),
            pltpu.SemaphoreType.DMA,
            pltpu.SemaphoreType.DMA,
            pltpu.SemaphoreType.DMA,
            pltpu.SemaphoreType.DMA,
        ],
        compiler_params=pltpu.CompilerParams(use_tc_tiling_on_sc=False),
    )
    def gk(table_hbm, idx_hbm, out_hbm, idx_all, buf0, buf1, gs0, gs1,
           ws0, ws1):
        wid = lax.axis_index("s") * NC + lax.axis_index("c")
        base_w = wid * per_w
        pltpu.sync_copy(idx_hbm.at[pl.ds(wid * r_rows, r_rows)], idx_all)

        def fire_gather(k, buf, sem):
            for j in range(rpc):
                pltpu.async_copy(
                    table_hbm.at[idx_all.at[k * rpc + j]],
                    buf.at[pl.ds(j * cj, cj)],
                    sem,
                )

        def wait_gather(buf, sem):
            for j in range(rpc):
                pltpu.make_async_copy(
                    table_hbm.at[idx_all.at[j]], buf.at[pl.ds(j * cj, cj)],
                    sem,
                ).wait()

        def wr(k, buf, sem):
            return pltpu.make_async_copy(
                buf, out_hbm.at[pl.ds(base_w + k * chunk, chunk)], sem
            )

        def body(i, _):
            k0 = 2 * i
            k1 = 2 * i + 1

            @pl.when(i >= 1)
            def _():
                wr(0, buf0, ws0).wait()

            fire_gather(k0, buf0, gs0)

            @pl.when(i >= 1)
            def _():
                wr(0, buf1, ws1).wait()

            fire_gather(k1, buf1, gs1)
            wait_gather(buf0, gs0)
            wr(k0, buf0, ws0).start()
            wait_gather(buf1, gs1)
            wr(k1, buf1, ws1).start()
            return 0

        lax.fori_loop(0, n2, body, 0)
        wr(0, buf0, ws0).wait()
        wr(0, buf1, ws1).wait()

    return gk(table, idx2)


def _sc_scatter_add(x, dst, n_out, dc, tiled=False):
    e, d = x.shape
    chunk = 1000 if tiled else 500
    rpc = 8 if tiled else 4
    cj = 125
    assert e % (NW * chunk) == 0 and d % dc == 0 and n_out % NS == 0
    per_w = e // NW
    r_rows = per_w // cj
    n_chunks = per_w // chunk
    odd = n_chunks % 2
    n2 = (n_chunks - odd) // 2
    n_pass = d // dc
    rows_per_sub = n_out // NS
    if tiled:
        assert rows_per_sub % 8 == 0 and chunk % 8 == 0

    dst2 = dst.reshape(e // cj, cj)
    mesh = plsc.VectorSubcoreMesh(core_axis_name="c", subcore_axis_name="s")

    @functools.partial(
        pl.kernel,
        mesh=mesh,
        out_type=jax.ShapeDtypeStruct((NC, n_out, d), jnp.float32),
        scratch_types=[
            pltpu.VMEM_SHARED((n_out, dc), jnp.float32),
            pltpu.VMEM((r_rows, cj), jnp.int32),
            pltpu.VMEM((chunk, dc), jnp.float32),
            pltpu.VMEM((chunk, dc), jnp.float32),
            pltpu.SemaphoreType.DMA,
            pltpu.SemaphoreType.DMA,
            pltpu.SemaphoreType.DMA,
            pltpu.SemaphoreType.DMA,
        ],
        compiler_params=pltpu.CompilerParams(use_tc_tiling_on_sc=tiled),
    )
    def sk(x_hbm, dst_hbm, zeros_hbm, out_hbm, acc, idx_all, xb0, xb1,
           xs0, xs1, ss0, ss1):
        cid = lax.axis_index("c")
        sid = lax.axis_index("s")
        row0 = sid * rows_per_sub
        wid = cid * NS + sid
        e0 = wid * per_w
        pltpu.sync_copy(dst_hbm.at[pl.ds(wid * r_rows, r_rows)], idx_all)

        def load_x(k, d0, buf, sem):
            pltpu.async_copy(
                x_hbm.at[pl.ds(e0 + k * chunk, chunk), pl.ds(d0, dc)],
                buf, sem,
            )

        def wait_x(d0, buf, sem):
            pltpu.make_async_copy(
                x_hbm.at[pl.ds(e0, chunk), pl.ds(d0, dc)], buf, sem
            ).wait()

        def fire_adds(k, buf, sem):
            for j in range(rpc):
                pltpu.async_copy(
                    buf.at[pl.ds(j * cj, cj)],
                    acc.at[idx_all.at[k * rpc + j]],
                    sem, add=True,
                )

        def wait_adds(buf, sem):
            for j in range(rpc):
                pltpu.make_async_copy(
                    buf.at[pl.ds(j * cj, cj)], acc.at[idx_all.at[j]], sem
                ).wait()

        def one_pass(p, _):
            d0 = p * dc
            pltpu.sync_copy(zeros_hbm, acc.at[pl.ds(row0, rows_per_sub)])
            plsc.subcore_barrier()

            if odd:
                load_x(0, d0, xb0, xs0)
                wait_x(d0, xb0, xs0)
                fire_adds(0, xb0, ss0)
                wait_adds(xb0, ss0)

            def chunk_body(i, _):
                k0 = odd + 2 * i
                k1 = k0 + 1

                @pl.when(i >= 1)
                def _():
                    wait_adds(xb0, ss0)

                load_x(k0, d0, xb0, xs0)

                @pl.when(i >= 1)
                def _():
                    wait_adds(xb1, ss1)

                load_x(k1, d0, xb1, xs1)
                wait_x(d0, xb0, xs0)
                fire_adds(k0, xb0, ss0)
                wait_x(d0, xb1, xs1)
                fire_adds(k1, xb1, ss1)
                return 0

            lax.fori_loop(0, n2, chunk_body, 0)
            wait_adds(xb0, ss0)
            wait_adds(xb1, ss1)
            plsc.subcore_barrier()
            pltpu.sync_copy(
                acc.at[pl.ds(row0, rows_per_sub)],
                out_hbm.at[cid, pl.ds(row0, rows_per_sub), pl.ds(d0, dc)],
            )
            plsc.subcore_barrier()
            return 0

        lax.fori_loop(0, n_pass, one_pass, 0)

    zeros = jnp.zeros((rows_per_sub, dc), jnp.float32)
    return sk(x, dst2, zeros)


def _row_specs(shapes, bn):
    specs = []
    for s, tiled in shapes:
        if tiled == "p":
            specs.append(
                pl.BlockSpec((s[0], bn, s[2]), lambda i: (0, i, 0))
            )
        elif tiled:
            blk = (bn,) + tuple(s[1:])
            nd = len(s)
            specs.append(
                pl.BlockSpec(blk, lambda i, _nd=nd: (i,) + (0,) * (_nd - 1))
            )
        else:
            specs.append(pl.BlockSpec(s, lambda i, _nd=len(s): (0,) * _nd))
    return specs


def _silu(x):
    return x * jax.nn.sigmoid(x)


def _tc_call(body, n_rows, bn, ins, in_tiled, out_shapes, out_tiled):
    grid = (n_rows // bn,)
    in_specs = _row_specs([(tuple(a.shape), t) for a, t in zip(ins, in_tiled)], bn)
    out_specs = _row_specs([(tuple(s.shape), t) for s, t in zip(out_shapes, out_tiled)], bn)
    return pl.pallas_call(
        body,
        grid=grid,
        in_specs=in_specs,
        out_specs=out_specs if len(out_shapes) > 1 else out_specs[0],
        out_shape=out_shapes if len(out_shapes) > 1 else out_shapes[0],
        compiler_params=pltpu.CompilerParams(
            dimension_semantics=("arbitrary",)
        ),
    )(*ins)


def _geometry(pos_src, pos_dst):
    e = pos_src.shape[0]
    bn = 2000

    def body(ps_ref, pd_ref, geo_ref):
        ps = ps_ref[...]
        pd = pd_ref[...]
        diff = pd - ps
        mask = lax.broadcasted_iota(jnp.int32, (1, 16), 1) < 3
        dm = jnp.where(mask, diff, 0.0)
        d2 = jnp.sum(dm * dm, axis=1, keepdims=True)
        dd = jnp.sqrt(d2 + 1e-12)
        unit = dm / (dd + 1e-10)
        fc = jnp.where(
            dd < CUTOFF, 0.5 * (jnp.cos(jnp.pi * dd / CUTOFF) + 1.0), 0.0
        )
        geo_ref[...] = jnp.concatenate(
            [unit[:, 0:3], dd, fc, jnp.zeros_like(dd), dd * 0.0, dd * 0.0],
            axis=1,
        )

    out = jax.ShapeDtypeStruct((e, 8), jnp.float32)
    return _tc_call(body, e, bn, [pos_src, pos_dst], [True, True], [out], [True])


def _node_msg(s, w1, b1):
    n = s.shape[0]
    bn = 2000

    def body(s_ref, w1_ref, b1_ref, o_ref):
        h = jnp.dot(s_ref[...], w1_ref[...], preferred_element_type=jnp.float32)
        o_ref[...] = _silu(h + b1_ref[...])

    out = jax.ShapeDtypeStruct((n, D_S), jnp.float32)
    return _tc_call(
        body, n, bn,
        [s, w1, b1], [True, False, False],
        [out], [True],
    )


def _edge_kernel(geo8, h_src, v_src, w2, b2, wf, bf):
    e = geo8.shape[0]
    bn = 2000

    def body(g_ref, h_ref, v_ref, w2_ref, b2_ref, wf_ref, bf_ref, o_ref):
        ns = lax.broadcasted_iota(jnp.int32, (1, 24), 1).astype(jnp.float32) + 1.0
        geo = g_ref[...]
        dd = geo[:, 3:4]
        fc = geo[:, 4:5]
        es = jnp.sin(dd * ns * (jnp.pi / CUTOFF)) / dd
        fw = jnp.dot(es, wf_ref[...], preferred_element_type=jnp.float32)
        fw = (fw + bf_ref[...]) * fc
        sm = jnp.dot(h_ref[...], w2_ref[...], preferred_element_type=jnp.float32)
        fo = fw * (sm + b2_ref[...])
        g_vec = fo[:, 0:64]
        g_edge = fo[:, 64:128]
        g_sca = fo[:, 128:192]
        v = v_ref[...]
        parts = [g_sca]
        for k in range(3):
            uk = geo[:, k:k + 1]
            parts.append(v[:, 64 * k:64 * k + 64] * g_vec + g_edge * uk)
        o_ref[...] = jnp.concatenate(parts, axis=1)

    out = jax.ShapeDtypeStruct((e, 256), jnp.float32)
    return _tc_call(
        body, e, bn,
        [geo8, h_src, v_src, w2, b2, wf, bf],
        [True, True, True, False, False, False, False],
        [out], [True],
    )


def _edge_kernel0(geo8, h_src, w2, b2, wf, bf):
    e = geo8.shape[0]
    bn = 2000

    def body(g_ref, h_ref, w2_ref, b2_ref, wf_ref, bf_ref, o_ref):
        ns = lax.broadcasted_iota(jnp.int32, (1, 24), 1).astype(jnp.float32) + 1.0
        geo = g_ref[...]
        dd = geo[:, 3:4]
        fc = geo[:, 4:5]
        es = jnp.sin(dd * ns * (jnp.pi / CUTOFF)) / dd
        fw = jnp.dot(es, wf_ref[...], preferred_element_type=jnp.float32)
        fw = (fw + bf_ref[...]) * fc
        sm = jnp.dot(h_ref[...], w2_ref[...], preferred_element_type=jnp.float32)
        fo = fw * (sm + b2_ref[...])
        g_edge = fo[:, 64:128]
        g_sca = fo[:, 128:192]
        parts = [g_sca]
        for k in range(3):
            parts.append(g_edge * geo[:, k:k + 1])
        o_ref[...] = jnp.concatenate(parts, axis=1)

    out = jax.ShapeDtypeStruct((e, 256), jnp.float32)
    return _tc_call(
        body, e, bn,
        [geo8, h_src, w2, b2, wf, bf],
        [True, True, False, False, False, False],
        [out], [True],
    )


def _node_update(s, v, partials, wu, wv, w1, b1, w2, b2):
    n = s.shape[0]
    bn = 2000

    def body(s_ref, v_ref, p_ref, wu_ref, wv_ref, w1_ref, b1_ref, w2_ref,
             b2_ref, so_ref, vo_ref):
        agg = p_ref[0] + p_ref[1]
        s1 = s_ref[...] + agg[:, 0:64]
        v1 = v_ref[...] + agg[:, 64:256]
        wu_m = wu_ref[...]
        wv_m = wv_ref[...]
        uv = []
        vv = []
        for k in range(3):
            vk = v1[:, 64 * k:64 * k + 64]
            uv.append(jnp.dot(vk, wu_m, preferred_element_type=jnp.float32))
            vv.append(jnp.dot(vk, wv_m, preferred_element_type=jnp.float32))
        vn2 = vv[0] * vv[0] + vv[1] * vv[1] + vv[2] * vv[2]
        vn = jnp.sqrt(vn2 + 1e-8)
        h = jnp.concatenate([s1, vn], axis=1)
        h = jnp.dot(h, w1_ref[...], preferred_element_type=jnp.float32)
        h = _silu(h + b1_ref[...])
        m = jnp.dot(h, w2_ref[...], preferred_element_type=jnp.float32)
        m = m + b2_ref[...]
        a_vv = m[:, 0:64]
        a_sv = m[:, 64:128]
        a_ss = m[:, 128:192]
        v2 = [v1[:, 64 * k:64 * k + 64] + a_vv * uv[k] for k in range(3)]
        inner = uv[0] * vv[0] + uv[1] * vv[1] + uv[2] * vv[2]
        so_ref[...] = s1 + a_sv * inner + a_ss
        vo_ref[...] = jnp.concatenate(v2, axis=1)

    outs = [
        jax.ShapeDtypeStruct((n, D_S), jnp.float32),
        jax.ShapeDtypeStruct((n, D_SM), jnp.float32),
    ]
    return _tc_call(
        body, n, bn,
        [s, v, partials, wu, wv, w1, b1, w2, b2],
        [True, True, "p", False, False, False, False, False, False],
        outs, [True, True],
    )


def _final_kernel(s, lap_partials, w1, b1, w2, b2):
    n = s.shape[0]
    bn = 2000

    def body(s_ref, p_ref, w1_ref, b1_ref, w2_ref, b2_ref, r_ref, lap_ref):
        sv = s_ref[...]
        h = jnp.dot(sv, w1_ref[...], preferred_element_type=jnp.float32)
        h = _silu(h + b1_ref[...])
        r = jnp.dot(h, w2_ref[...], preferred_element_type=jnp.float32)
        r_ref[...] = r + b2_ref[...]
        agg = p_ref[0] + p_ref[1]
        segsum = agg[:, 0:64]
        deg = agg[:, 64:65]
        lap_ref[...] = deg * sv - segsum

    outs = [
        jax.ShapeDtypeStruct((n, 1), jnp.float32),
        jax.ShapeDtypeStruct((n, D_S), jnp.float32),
    ]
    return _tc_call(
        body, n, bn,
        [s, lap_partials, w1, b1, w2, b2],
        [True, "p", False, False, False, False],
        outs, [True, True],
    )


def kernel(node_features, node_positions, edge_index, params):
    n = node_positions.shape[0]
    e = edge_index.shape[1]
    src = edge_index[0].astype(jnp.int32)
    dst = edge_index[1].astype(jnp.int32)

    n_pad = 50176
    nf = jnp.pad(node_features.astype(jnp.int32), (0, n_pad - n))
    s = _sc_gather(params["embedding"], nf, cj=112, rpc=7)[:n]

    pos16 = jnp.pad(node_positions, ((0, 0), (0, 13)))
    p_src = _sc_gather(pos16, src, cj=125, rpc=4)
    p_dst = _sc_gather(pos16, dst, cj=125, rpc=4)
    geo8 = _geometry(p_src, p_dst)

    v = jnp.zeros((n, D_SM), jnp.float32)

    for bi, bp in enumerate(params["blocks"]):
        wf = jnp.pad(bp["filter"]["w"], ((0, 4), (0, 0)))
        bf = bp["filter"]["b"].reshape(1, D_SM)
        w2 = bp["msg2"]["w"]
        b2 = bp["msg2"]["b"].reshape(1, D_SM)
        h = _node_msg(s, bp["msg1"]["w"], bp["msg1"]["b"].reshape(1, D_S))
        h_src = _sc_gather(h, src, cj=125, rpc=4)
        if bi == 0:
            scat_in = _edge_kernel0(geo8, h_src, w2, b2, wf, bf)
        else:
            v_src = _sc_gather(v, src, cj=125, rpc=2)
            scat_in = _edge_kernel(geo8, h_src, v_src, w2, b2, wf, bf)
        partials = _sc_scatter_add(scat_in, dst, n, dc=16)
        s, v = _node_update(
            s, v, partials,
            bp["U"]["w"], bp["V"]["w"],
            bp["upd1"]["w"], bp["upd1"]["b"].reshape(1, D_S),
            bp["upd2"]["w"], bp["upd2"]["b"].reshape(1, D_SM),
        )

    s_ext = jnp.concatenate([s, jnp.ones((n, 16), jnp.float32)], axis=1)
    sd = _sc_gather(s_ext, dst, cj=125, rpc=4)
    lap_partials = _sc_scatter_add(sd, src, n, dc=16)

    r0, r1 = params["readout"]
    r, lap = _final_kernel(
        s, lap_partials,
        r0["w"], r0["b"].reshape(1, D_S),
        r1["w"], r1["b"].reshape(1, 1),
    )
    return (r, s, lap)
